# Initial kernel scaffold; baseline (speedup 1.0000x reference)
import jax
import jax.numpy as jnp
from jax.experimental import pallas as pl


def kernel(x, edge_index, edge_attr, batch, global_feat, params):
    raise NotImplementedError("write your pallas kernel here")


# speedup vs baseline: 2.9525x; 2.9525x over previous
import jax
import jax.numpy as jnp
from jax import lax
from jax.experimental import pallas as pl
from jax.experimental.pallas import tpu as pltpu
from jax.experimental.pallas import tpu_sc as plsc

N = 10000
E = 320000
G = 64
DE = 16
HID = 128

NC = 2
NS = 16
NW = NC * NS
EPW = E // NW
K = 80
NCHUNK = EPW // K
AR = 80
NAC = N // AR

f32 = jnp.float32


def _mp_body(h_hbm, e_hbm, src_hbm, dst_hbm, out0_hbm, out1_hbm,
             src_v, dst_v, hrow_v, erow_v, acc_sh, sem):
    cid = lax.axis_index("c")
    sid = lax.axis_index("s")
    wid = sid * NC + cid

    def _zrow(r, carry):
        for j in range(8):
            hrow_v[r, pl.ds(j * 16, 16)] = jnp.zeros((16,), f32)
        return carry
    lax.fori_loop(0, AR, _zrow, 0)

    def _zacc(j, carry):
        idx = sid + NS * j
        @pl.when(idx < NAC)
        def _():
            pltpu.sync_copy(hrow_v, acc_sh.at[pl.ds(idx * AR, AR)])
        return carry
    lax.fori_loop(0, (NAC + NS - 1) // NS, _zacc, 0)

    pltpu.sync_copy(src_hbm.at[pl.ds(wid * EPW, EPW)], src_v)
    pltpu.sync_copy(dst_hbm.at[wid], dst_v)

    plsc.subcore_barrier()

    def _chunk(c, carry):
        base = wid * EPW + c * K
        pltpu.sync_copy(e_hbm.at[pl.ds(base, K)], erow_v)
        pltpu.async_copy(h_hbm.at[src_v.at[pl.ds(c * K, K)]], hrow_v, sem).wait()

        def _row(r, rc):
            for j in range(8):
                s = pl.ds(j * 16, 16)
                erow_v[r, s] = jnp.maximum(erow_v[r, s] + hrow_v[r, s], 0.0)
            return rc
        lax.fori_loop(0, K, _row, 0)

        pltpu.sync_copy(erow_v, acc_sh.at[dst_v.at[c]], add=True)
        return carry
    lax.fori_loop(0, NCHUNK, _chunk, 0)

    plsc.subcore_barrier()

    def _wout(j, carry):
        idx = sid + NS * j
        @pl.when(idx < NAC)
        def _():
            sl = pl.ds(idx * AR, AR)
            @pl.when(cid == 0)
            def _():
                pltpu.sync_copy(acc_sh.at[sl], out0_hbm.at[sl])
            @pl.when(cid == 1)
            def _():
                pltpu.sync_copy(acc_sh.at[sl], out1_hbm.at[sl])
        return carry
    lax.fori_loop(0, (NAC + NS - 1) // NS, _wout, 0)


_mp_call = pl.kernel(
    _mp_body,
    out_type=(jax.ShapeDtypeStruct((N, HID), f32),
              jax.ShapeDtypeStruct((N, HID), f32)),
    mesh=plsc.VectorSubcoreMesh(core_axis_name="c", subcore_axis_name="s"),
    scratch_types=[
        pltpu.VMEM((EPW,), jnp.int32),
        pltpu.VMEM((NCHUNK, K), jnp.int32),
        pltpu.VMEM((K, HID), f32),
        pltpu.VMEM((K, HID), f32),
        pltpu.VMEM_SHARED((N, HID), f32),
        pltpu.SemaphoreType.DMA,
    ],
)


EB = 2000


def _edge_body(ea_ref, w_ref, b_ref, e0_ref, e1_ref, e2_ref):
    out = jnp.dot(ea_ref[...], w_ref[...], preferred_element_type=f32) + b_ref[...]
    e0_ref[...] = out[:, 0:HID]
    e1_ref[...] = out[:, HID:2 * HID]
    e2_ref[...] = out[:, 2 * HID:3 * HID]


_edge_call = pl.pallas_call(
    _edge_body,
    grid=(E // EB,),
    in_specs=[
        pl.BlockSpec((EB, DE), lambda i: (i, 0)),
        pl.BlockSpec((DE, 3 * HID), lambda i: (0, 0)),
        pl.BlockSpec((1, 3 * HID), lambda i: (0, 0)),
    ],
    out_specs=[pl.BlockSpec((EB, HID), lambda i: (i, 0))] * 3,
    out_shape=[jax.ShapeDtypeStruct((E, HID), f32)] * 3,
)

NB = 2000


def _mlp_body(h_ref, a0_ref, a1_ref, w1_ref, b1_ref, w2_ref, b2_ref,
              g_ref, bt_ref, rm_ref, rv_ref, o_ref):
    h2 = h_ref[...] + a0_ref[...] + a1_ref[...]
    a = jnp.maximum(jnp.dot(h2, w1_ref[...], preferred_element_type=f32) + b1_ref[...], 0.0)
    o = jnp.dot(a, w2_ref[...], preferred_element_type=f32) + b2_ref[...]
    sc = g_ref[...] * lax.rsqrt(rv_ref[...] + 1e-5)
    o_ref[...] = jnp.maximum((o - rm_ref[...]) * sc + bt_ref[...], 0.0)


_mlp_call = pl.pallas_call(
    _mlp_body,
    grid=(N // NB,),
    in_specs=[pl.BlockSpec((NB, HID), lambda i: (i, 0))] * 3
    + [
        pl.BlockSpec((HID, HID), lambda i: (0, 0)),
        pl.BlockSpec((1, HID), lambda i: (0, 0)),
        pl.BlockSpec((HID, HID), lambda i: (0, 0)),
        pl.BlockSpec((1, HID), lambda i: (0, 0)),
        pl.BlockSpec((1, HID), lambda i: (0, 0)),
        pl.BlockSpec((1, HID), lambda i: (0, 0)),
        pl.BlockSpec((1, HID), lambda i: (0, 0)),
        pl.BlockSpec((1, HID), lambda i: (0, 0)),
    ],
    out_specs=pl.BlockSpec((NB, HID), lambda i: (i, 0)),
    out_shape=jax.ShapeDtypeStruct((N, HID), f32),
)

PB = 2000


def _pool_body(h_ref, b_ref, gf_ref, wp1_ref, bp1_ref, wp2_ref, bp2_ref,
               wfa_ref, wfb_ref, bf_ref, wm_ref, bm_ref,
               mu_ref, z_ref, g_ref):
    i = pl.program_id(0)
    bat = b_ref[0]
    ids = lax.broadcasted_iota(jnp.int32, (G, PB), 0)
    mask = (bat == ids).astype(f32)
    part = jnp.dot(mask, h_ref[...], preferred_element_type=f32)

    @pl.when(i == 0)
    def _():
        g_ref[...] = part

    @pl.when(i > 0)
    def _():
        g_ref[...] = g_ref[...] + part

    @pl.when(i == pl.num_programs(0) - 1)
    def _():
        g = g_ref[...]
        p1 = jnp.maximum(jnp.dot(g, wp1_ref[...], preferred_element_type=f32) + bp1_ref[...], 0.0)
        z_ref[...] = jax.nn.sigmoid(jnp.dot(p1, wp2_ref[...], preferred_element_type=f32) + bp2_ref[...])
        gfin = jnp.maximum(
            jnp.dot(g, wfa_ref[...], preferred_element_type=f32)
            + jnp.dot(gf_ref[...], wfb_ref[...], preferred_element_type=f32)
            + bf_ref[...], 0.0)
        mu_ref[...] = jnp.sum(gfin * wm_ref[...], axis=1, keepdims=True) + bm_ref[...]


_pool_call = pl.pallas_call(
    _pool_body,
    grid=(N // PB,),
    in_specs=[
        pl.BlockSpec((PB, HID), lambda i: (i, 0)),
        pl.BlockSpec((1, 1, PB), lambda i: (i, 0, 0)),
        pl.BlockSpec((G, 2), lambda i: (0, 0)),
        pl.BlockSpec((HID, G), lambda i: (0, 0)),
        pl.BlockSpec((1, G), lambda i: (0, 0)),
        pl.BlockSpec((G, HID), lambda i: (0, 0)),
        pl.BlockSpec((1, HID), lambda i: (0, 0)),
        pl.BlockSpec((HID, HID), lambda i: (0, 0)),
        pl.BlockSpec((2, HID), lambda i: (0, 0)),
        pl.BlockSpec((1, HID), lambda i: (0, 0)),
        pl.BlockSpec((1, HID), lambda i: (0, 0)),
        pl.BlockSpec((1, 1), lambda i: (0, 0)),
    ],
    out_specs=[
        pl.BlockSpec((G, 1), lambda i: (0, 0)),
        pl.BlockSpec((G, HID), lambda i: (0, 0)),
        pl.BlockSpec((G, HID), lambda i: (0, 0)),
    ],
    out_shape=[
        jax.ShapeDtypeStruct((G, 1), f32),
        jax.ShapeDtypeStruct((G, HID), f32),
        jax.ShapeDtypeStruct((G, HID), f32),
    ],
)


def kernel(x, edge_index, edge_attr, batch, global_feat, params):
    src = edge_index[0]
    dst_r = edge_index[1].reshape(NW, NCHUNK, K)
    lps = params["layers"]

    wcat = jnp.concatenate([lp["eW"].T for lp in lps], axis=1)
    bcat = jnp.concatenate([lp["eb"] for lp in lps]).reshape(1, 3 * HID)
    es = _edge_call(edge_attr, wcat, bcat)

    h = x
    for l, lp in enumerate(lps):
        a0, a1 = _mp_call(h, es[l], src, dst_r)
        h = _mlp_call(
            h, a0, a1,
            lp["W1"].T, lp["b1"].reshape(1, HID),
            lp["W2"].T, lp["b2"].reshape(1, HID),
            lp["g"].reshape(1, HID), lp["bt"].reshape(1, HID),
            lp["rm"].reshape(1, HID), lp["rv"].reshape(1, HID),
        )

    batr = batch.reshape(N // PB, 1, PB)
    mu, z, g = _pool_call(
        h, batr, global_feat,
        params["Wp1"].T, params["bp1"].reshape(1, G),
        params["Wp2"].T, params["bp2"].reshape(1, HID),
        params["Wf"][:, :HID].T, params["Wf"][:, HID:].T,
        params["bf"].reshape(1, HID),
        params["Wm"].reshape(1, HID), params["bm"].reshape(1, 1),
    )
    return (mu.reshape(-1), z, g)


# speedup vs baseline: 3.4830x; 1.1797x over previous
import jax
import jax.numpy as jnp
from jax import lax
from jax.experimental import pallas as pl
from jax.experimental.pallas import tpu as pltpu
from jax.experimental.pallas import tpu_sc as plsc

N = 10000
E = 320000
G = 64
DE = 16
HID = 128

NC = 2
NS = 16
NW = NC * NS
EPW = E // NW
K = 40
NCHUNK = EPW // K
NBUF = 3
NGRP = (NCHUNK + NBUF - 1) // NBUF
AR = K
NAC = N // AR

f32 = jnp.float32


def _mp_body(h_hbm, e_hbm, src_hbm, dst_hbm, out0_hbm, out1_hbm,
             s0, s1, s2, d0, d1, d2, hb0, hb1, hb2, eb0, eb1, eb2,
             l0, l1, l2, g0, g1, g2, x0, x1, x2, acc_sh):
    srcs, dsts = (s0, s1, s2), (d0, d1, d2)
    hrows, erows = (hb0, hb1, hb2), (eb0, eb1, eb2)
    lsem, gsem, ssem = (l0, l1, l2), (g0, g1, g2), (x0, x1, x2)

    cid = lax.axis_index("c")
    sid = lax.axis_index("s")
    wid = sid * NC + cid
    base0 = wid * EPW

    def _zrow(r, carry):
        for j in range(8):
            hb0[r, pl.ds(j * 16, 16)] = jnp.zeros((16,), f32)
        return carry
    lax.fori_loop(0, AR, _zrow, 0)

    def _zacc(j, carry):
        idx = sid + NS * j
        @pl.when(idx < NAC)
        def _():
            pltpu.sync_copy(hb0, acc_sh.at[pl.ds(idx * AR, AR)])
        return carry
    lax.fori_loop(0, (NAC + NS - 1) // NS, _zacc, 0)
    plsc.subcore_barrier()

    def issue_loads(c, b):
        base = base0 + c * K
        pltpu.async_copy(src_hbm.at[pl.ds(base, K)], srcs[b], lsem[b])
        pltpu.async_copy(dst_hbm.at[pl.ds(base, K)], dsts[b], lsem[b])
        pltpu.async_copy(e_hbm.at[pl.ds(base, K)], erows[b], gsem[b])

    def wait_idx(b):
        pltpu.make_async_copy(src_hbm.at[pl.ds(0, K)], srcs[b], lsem[b]).wait()
        pltpu.make_async_copy(src_hbm.at[pl.ds(0, K)], dsts[b], lsem[b]).wait()

    def issue_gather(b):
        pltpu.async_copy(h_hbm.at[srcs[b]], hrows[b], gsem[b])

    def wait_gather_e(b):
        pltpu.make_async_copy(e_hbm.at[pl.ds(0, K)], erows[b], gsem[b]).wait()
        pltpu.make_async_copy(e_hbm.at[pl.ds(0, K)], hrows[b], gsem[b]).wait()

    def compute(b):
        er, hr = erows[b], hrows[b]
        def _row(r, rc):
            for j in range(8):
                s = pl.ds(j * 16, 16)
                er[r, s] = jnp.maximum(er[r, s] + hr[r, s], 0.0)
            return rc
        lax.fori_loop(0, K, _row, 0)

    def issue_scatter(b):
        pltpu.async_copy(erows[b], acc_sh.at[dsts[b]], ssem[b], add=True)

    def wait_scatter(b):
        pltpu.make_async_copy(erows[b], acc_sh.at[pl.ds(0, K)], ssem[b]).wait()

    issue_loads(0, 0)
    issue_loads(1, 1)
    wait_idx(0)
    issue_gather(0)

    def _grp(p, carry):
        c0 = NBUF * p
        for b in range(NBUF):
            c = c0 + b
            @pl.when(c < NCHUNK)
            def _(c=c, b=b):
                wait_gather_e(b)
                compute(b)
                issue_scatter(b)
                @pl.when(c + 1 < NCHUNK)
                def _(c=c, b=b):
                    wait_idx((b + 1) % NBUF)
                    issue_gather((b + 1) % NBUF)
                @pl.when(c + 2 < NCHUNK)
                def _(c=c, b=b):
                    @pl.when(c >= 1)
                    def _(b=b):
                        wait_scatter((b + 2) % NBUF)
                    issue_loads(c + 2, (b + 2) % NBUF)
        return carry
    lax.fori_loop(0, NGRP, _grp, 0)

    for b in range(NBUF):
        wait_scatter(b)

    plsc.subcore_barrier()

    def _wout(j, carry):
        idx = sid + NS * j
        @pl.when(idx < NAC)
        def _():
            sl = pl.ds(idx * AR, AR)
            @pl.when(cid == 0)
            def _():
                pltpu.sync_copy(acc_sh.at[sl], out0_hbm.at[sl])
            @pl.when(cid == 1)
            def _():
                pltpu.sync_copy(acc_sh.at[sl], out1_hbm.at[sl])
        return carry
    lax.fori_loop(0, (NAC + NS - 1) // NS, _wout, 0)


_mp_scratch = (
    [pltpu.VMEM((K,), jnp.int32)] * 6
    + [pltpu.VMEM((K, HID), f32)] * 6
    + [pltpu.SemaphoreType.DMA] * 9
    + [pltpu.VMEM_SHARED((N, HID), f32)]
)

_mp_call = pl.kernel(
    _mp_body,
    out_type=(jax.ShapeDtypeStruct((N, HID), f32),
              jax.ShapeDtypeStruct((N, HID), f32)),
    mesh=plsc.VectorSubcoreMesh(core_axis_name="c", subcore_axis_name="s"),
    scratch_types=_mp_scratch,
)


EB = 2000


def _edge_body(ea_ref, w_ref, b_ref, e0_ref, e1_ref, e2_ref):
    out = jnp.dot(ea_ref[...], w_ref[...], preferred_element_type=f32) + b_ref[...]
    e0_ref[...] = out[:, 0:HID]
    e1_ref[...] = out[:, HID:2 * HID]
    e2_ref[...] = out[:, 2 * HID:3 * HID]


_edge_call = pl.pallas_call(
    _edge_body,
    grid=(E // EB,),
    in_specs=[
        pl.BlockSpec((EB, DE), lambda i: (i, 0)),
        pl.BlockSpec((DE, 3 * HID), lambda i: (0, 0)),
        pl.BlockSpec((1, 3 * HID), lambda i: (0, 0)),
    ],
    out_specs=[pl.BlockSpec((EB, HID), lambda i: (i, 0))] * 3,
    out_shape=[jax.ShapeDtypeStruct((E, HID), f32)] * 3,
)

NB = 2000


def _mlp_body(h_ref, a0_ref, a1_ref, w1_ref, b1_ref, w2_ref, b2_ref,
              g_ref, bt_ref, rm_ref, rv_ref, o_ref):
    h2 = h_ref[...] + a0_ref[...] + a1_ref[...]
    a = jnp.maximum(jnp.dot(h2, w1_ref[...], preferred_element_type=f32) + b1_ref[...], 0.0)
    o = jnp.dot(a, w2_ref[...], preferred_element_type=f32) + b2_ref[...]
    sc = g_ref[...] * lax.rsqrt(rv_ref[...] + 1e-5)
    o_ref[...] = jnp.maximum((o - rm_ref[...]) * sc + bt_ref[...], 0.0)


_mlp_call = pl.pallas_call(
    _mlp_body,
    grid=(N // NB,),
    in_specs=[pl.BlockSpec((NB, HID), lambda i: (i, 0))] * 3
    + [
        pl.BlockSpec((HID, HID), lambda i: (0, 0)),
        pl.BlockSpec((1, HID), lambda i: (0, 0)),
        pl.BlockSpec((HID, HID), lambda i: (0, 0)),
        pl.BlockSpec((1, HID), lambda i: (0, 0)),
        pl.BlockSpec((1, HID), lambda i: (0, 0)),
        pl.BlockSpec((1, HID), lambda i: (0, 0)),
        pl.BlockSpec((1, HID), lambda i: (0, 0)),
        pl.BlockSpec((1, HID), lambda i: (0, 0)),
    ],
    out_specs=pl.BlockSpec((NB, HID), lambda i: (i, 0)),
    out_shape=jax.ShapeDtypeStruct((N, HID), f32),
)

PB = 2000


def _pool_body(h_ref, b_ref, gf_ref, wp1_ref, bp1_ref, wp2_ref, bp2_ref,
               wfa_ref, wfb_ref, bf_ref, wm_ref, bm_ref,
               mu_ref, z_ref, g_ref):
    i = pl.program_id(0)
    bat = b_ref[0]
    ids = lax.broadcasted_iota(jnp.int32, (G, PB), 0)
    mask = (bat == ids).astype(f32)
    part = jnp.dot(mask, h_ref[...], preferred_element_type=f32)

    @pl.when(i == 0)
    def _():
        g_ref[...] = part

    @pl.when(i > 0)
    def _():
        g_ref[...] = g_ref[...] + part

    @pl.when(i == pl.num_programs(0) - 1)
    def _():
        g = g_ref[...]
        p1 = jnp.maximum(jnp.dot(g, wp1_ref[...], preferred_element_type=f32) + bp1_ref[...], 0.0)
        z_ref[...] = jax.nn.sigmoid(jnp.dot(p1, wp2_ref[...], preferred_element_type=f32) + bp2_ref[...])
        gfin = jnp.maximum(
            jnp.dot(g, wfa_ref[...], preferred_element_type=f32)
            + jnp.dot(gf_ref[...], wfb_ref[...], preferred_element_type=f32)
            + bf_ref[...], 0.0)
        mu_ref[...] = jnp.sum(gfin * wm_ref[...], axis=1, keepdims=True) + bm_ref[...]


_pool_call = pl.pallas_call(
    _pool_body,
    grid=(N // PB,),
    in_specs=[
        pl.BlockSpec((PB, HID), lambda i: (i, 0)),
        pl.BlockSpec((1, 1, PB), lambda i: (i, 0, 0)),
        pl.BlockSpec((G, 2), lambda i: (0, 0)),
        pl.BlockSpec((HID, G), lambda i: (0, 0)),
        pl.BlockSpec((1, G), lambda i: (0, 0)),
        pl.BlockSpec((G, HID), lambda i: (0, 0)),
        pl.BlockSpec((1, HID), lambda i: (0, 0)),
        pl.BlockSpec((HID, HID), lambda i: (0, 0)),
        pl.BlockSpec((2, HID), lambda i: (0, 0)),
        pl.BlockSpec((1, HID), lambda i: (0, 0)),
        pl.BlockSpec((1, HID), lambda i: (0, 0)),
        pl.BlockSpec((1, 1), lambda i: (0, 0)),
    ],
    out_specs=[
        pl.BlockSpec((G, 1), lambda i: (0, 0)),
        pl.BlockSpec((G, HID), lambda i: (0, 0)),
        pl.BlockSpec((G, HID), lambda i: (0, 0)),
    ],
    out_shape=[
        jax.ShapeDtypeStruct((G, 1), f32),
        jax.ShapeDtypeStruct((G, HID), f32),
        jax.ShapeDtypeStruct((G, HID), f32),
    ],
)


def kernel(x, edge_index, edge_attr, batch, global_feat, params):
    src = edge_index[0]
    dst_r = edge_index[1]
    lps = params["layers"]

    wcat = jnp.concatenate([lp["eW"].T for lp in lps], axis=1)
    bcat = jnp.concatenate([lp["eb"] for lp in lps]).reshape(1, 3 * HID)
    es = _edge_call(edge_attr, wcat, bcat)

    h = x
    for l, lp in enumerate(lps):
        a0, a1 = _mp_call(h, es[l], src, dst_r)
        h = _mlp_call(
            h, a0, a1,
            lp["W1"].T, lp["b1"].reshape(1, HID),
            lp["W2"].T, lp["b2"].reshape(1, HID),
            lp["g"].reshape(1, HID), lp["bt"].reshape(1, HID),
            lp["rm"].reshape(1, HID), lp["rv"].reshape(1, HID),
        )

    batr = batch.reshape(N // PB, 1, PB)
    mu, z, g = _pool_call(
        h, batr, global_feat,
        params["Wp1"].T, params["bp1"].reshape(1, G),
        params["Wp2"].T, params["bp2"].reshape(1, HID),
        params["Wf"][:, :HID].T, params["Wf"][:, HID:].T,
        params["bf"].reshape(1, HID),
        params["Wm"].reshape(1, HID), params["bm"].reshape(1, 1),
    )
    return (mu.reshape(-1), z, g)
